# baseline (speedup 1.0000x reference)
import functools

import jax
import jax.numpy as jnp
from jax import lax
from jax.experimental import pallas as pl
from jax.experimental.pallas import tpu as pltpu
from jax.experimental.pallas import tpu_sc as plsc

MAX_CTX = 32768
DIM = 2048

_HALF_ROWS = MAX_CTX // 2
_WORKERS = 32
_PER_W = MAX_CTX // _WORKERS
_CH = 16
_NCH = _PER_W // _CH

_mesh = plsc.VectorSubcoreMesh(core_axis_name="c", subcore_axis_name="s")


@functools.partial(
    pl.kernel,
    out_type=jax.ShapeDtypeStruct((MAX_CTX, DIM), jnp.float32),
    mesh=_mesh,
    scratch_types=[
        pltpu.VMEM((2, _CH, DIM), jnp.float32),
        pltpu.SemaphoreType.DMA((2,)),
        pltpu.SemaphoreType.DMA((2,)),
    ],
)
def _sc_copy(mem_hbm, h_hbm, out_hbm, buf, rsem, wsem):
    wid = lax.axis_index("c") * 16 + lax.axis_index("s")
    base = wid * _PER_W

    def copy_stripe(src_ref, src_base):
        def read(c, slot):
            return pltpu.make_async_copy(
                src_ref.at[pl.ds(src_base + c * _CH, _CH), :],
                buf.at[slot], rsem.at[slot])

        def write(c, slot):
            return pltpu.make_async_copy(
                buf.at[slot],
                out_hbm.at[pl.ds(base + c * _CH, _CH), :], wsem.at[slot])

        read(0, 0).start()

        def step(c, _):
            slot = lax.rem(c, 2)
            nslot = lax.rem(c + 1, 2)
            read(c, slot).wait()

            @pl.when(c >= 1)
            def _():
                write(c - 1, nslot).wait()

            @pl.when(c + 1 < _NCH)
            def _():
                read(c + 1, nslot).start()

            write(c, slot).start()
            return 0

        lax.fori_loop(0, _NCH, step, 0)
        write(_NCH - 1, (_NCH - 1) % 2, ).wait()

    @pl.when(wid < _WORKERS // 2)
    def _():
        copy_stripe(mem_hbm, base + _HALF_ROWS)

    @pl.when(wid >= _WORKERS // 2)
    def _():
        copy_stripe(h_hbm, base - _HALF_ROWS)


def kernel(h, mem):
    B, L, D = h.shape
    flat = h.reshape(B * L, D)
    new_mem = _sc_copy(mem, flat)
    return h, new_mem
